# Initial kernel scaffold; baseline (speedup 1.0000x reference)
import jax
import jax.numpy as jnp
from jax.experimental import pallas as pl


def kernel(features_8, edge_index_8, edge_weight_8, features_9, edge_index_9, edge_weight_9):
    raise NotImplementedError("write your pallas kernel here")


# speedup vs baseline: 3.7580x; 3.7580x over previous
import dataclasses
import functools

import jax
import jax.numpy as jnp
from jax import lax
from jax.experimental import pallas as pl
from jax.experimental.pallas import tpu as pltpu
from jax.experimental.pallas import tpu_sc as plsc

L = 16
NW = 32
D = 128
E8 = 320000
E9 = 160000
PER_W8 = E8 // NW
PER_W9 = E9 // NW
W8 = 80
W9 = 40
NB8 = PER_W8 // W8
NB9 = PER_W9 // W9

_mesh = plsc.VectorSubcoreMesh(core_axis_name="c", subcore_axis_name="s")

_cp = pltpu.CompilerParams()
if "needs_layout_passes" in pltpu.CompilerParams.__dataclass_fields__:
    _cp = dataclasses.replace(_cp, needs_layout_passes=False)


def _accumulate(f_hbm, sidx_v, tidx_v, w_v, src_v, tgt_v, n_blocks, w_blk, scale, acc0):
    def block_body(b, acc):
        off = b * w_blk
        pltpu.sync_copy(f_hbm.at[sidx_v.at[pl.ds(off, w_blk)]], src_v)
        pltpu.sync_copy(f_hbm.at[tidx_v.at[pl.ds(off, w_blk)]], tgt_v)

        def edge_body(e, a):
            wb = plsc.load_gather(w_v, [jnp.full((L,), off + e, jnp.int32)])
            wb = wb * scale
            for c in range(D // L):
                s = src_v[e, pl.ds(c * L, L)]
                t = tgt_v[e, pl.ds(c * L, L)]
                d = s - t
                a = a + d * d * wb
            return a

        return lax.fori_loop(0, w_blk, edge_body, acc)

    return lax.fori_loop(0, n_blocks, block_body, acc0)


@functools.partial(
    pl.kernel,
    out_type=jax.ShapeDtypeStruct((NW, L), jnp.float32),
    mesh=_mesh,
    compiler_params=_cp,
    scratch_types=[
        pltpu.VMEM((PER_W8,), jnp.int32),
        pltpu.VMEM((PER_W8,), jnp.int32),
        pltpu.VMEM((PER_W8,), jnp.float32),
        pltpu.VMEM((PER_W9,), jnp.int32),
        pltpu.VMEM((PER_W9,), jnp.int32),
        pltpu.VMEM((PER_W9,), jnp.float32),
        pltpu.VMEM((W8, D), jnp.float32),
        pltpu.VMEM((W8, D), jnp.float32),
        pltpu.VMEM((W9, D), jnp.float32),
        pltpu.VMEM((W9, D), jnp.float32),
        pltpu.VMEM((L,), jnp.float32),
    ],
)
def _sc_loss(f8_hbm, s8_hbm, t8_hbm, w8_hbm, f9_hbm, s9_hbm, t9_hbm, w9_hbm,
             out_hbm,
             s8_v, t8_v, w8_v, s9_v, t9_v, w9_v,
             src8_v, tgt8_v, src9_v, tgt9_v, part_v):
    wid = lax.axis_index("c") * 16 + lax.axis_index("s")
    base8 = wid * PER_W8
    base9 = wid * PER_W9

    pltpu.sync_copy(s8_hbm.at[pl.ds(base8, PER_W8)], s8_v)
    pltpu.sync_copy(t8_hbm.at[pl.ds(base8, PER_W8)], t8_v)
    pltpu.sync_copy(w8_hbm.at[pl.ds(base8, PER_W8)], w8_v)
    pltpu.sync_copy(s9_hbm.at[pl.ds(base9, PER_W9)], s9_v)
    pltpu.sync_copy(t9_hbm.at[pl.ds(base9, PER_W9)], t9_v)
    pltpu.sync_copy(w9_hbm.at[pl.ds(base9, PER_W9)], w9_v)

    acc = jnp.zeros((L,), jnp.float32)
    acc = _accumulate(f8_hbm, s8_v, t8_v, w8_v, src8_v, tgt8_v,
                      NB8, W8, 1.0 / (2.0 * E8), acc)
    acc = _accumulate(f9_hbm, s9_v, t9_v, w9_v, src9_v, tgt9_v,
                      NB9, W9, 1.0 / (2.0 * E9), acc)

    part_v[...] = acc
    pltpu.sync_copy(part_v, out_hbm.at[wid])


def kernel(features_8, edge_index_8, edge_weight_8,
           features_9, edge_index_9, edge_weight_9):
    parts = _sc_loss(features_8, edge_index_8[0], edge_index_8[1], edge_weight_8,
                     features_9, edge_index_9[0], edge_index_9[1], edge_weight_9)
    return jnp.sum(parts)


# speedup vs baseline: 7.3426x; 1.9538x over previous
import dataclasses
import functools

import jax
import jax.numpy as jnp
from jax import lax
from jax.experimental import pallas as pl
from jax.experimental.pallas import tpu as pltpu
from jax.experimental.pallas import tpu_sc as plsc

L = 16
NW = 32
D = 128
E8 = 320000
E9 = 160000
PER_W8 = E8 // NW
PER_W9 = E9 // NW
W8 = 80
W9 = 40
NB8 = PER_W8 // W8
NB9 = PER_W9 // W9

_mesh = plsc.VectorSubcoreMesh(core_axis_name="c", subcore_axis_name="s")

_cp = pltpu.CompilerParams()
if "needs_layout_passes" in pltpu.CompilerParams.__dataclass_fields__:
    _cp = dataclasses.replace(_cp, needs_layout_passes=False)


def _accumulate(f_hbm, sidx_v, tidx_v, w_v, bufs, sems, n_blocks, w_blk, scale, acc0):
    assert n_blocks % 2 == 1

    def copies(b, k):
        off = b * w_blk
        return (
            pltpu.make_async_copy(
                f_hbm.at[sidx_v.at[pl.ds(off, w_blk)]], bufs[k][0], sems[k][0]),
            pltpu.make_async_copy(
                f_hbm.at[tidx_v.at[pl.ds(off, w_blk)]], bufs[k][1], sems[k][1]),
        )

    def start(b, k):
        for c in copies(b, k):
            c.start()

    def wait(b, k):
        for c in copies(b, k):
            c.wait()

    def compute(b, k, acc):
        src_v, tgt_v = bufs[k]
        off = b * w_blk

        def edge_body(e, a):
            wb = plsc.load_gather(w_v, [jnp.full((L,), off + e, jnp.int32)])
            wb = wb * scale
            for c in range(D // L):
                s = src_v[e, pl.ds(c * L, L)]
                t = tgt_v[e, pl.ds(c * L, L)]
                d = s - t
                a = a + d * d * wb
            return a

        return lax.fori_loop(0, w_blk, edge_body, acc)

    start(0, 0)

    def body2(i, acc):
        b = i * 2
        wait(b, 0)
        start(b + 1, 1)
        acc = compute(b, 0, acc)
        wait(b + 1, 1)
        start(b + 2, 0)
        return compute(b + 1, 1, acc)

    acc = lax.fori_loop(0, (n_blocks - 1) // 2, body2, acc0)
    b_last = n_blocks - 1
    wait(b_last, 0)
    return compute(b_last, 0, acc)


@functools.partial(
    pl.kernel,
    out_type=jax.ShapeDtypeStruct((NW, L), jnp.float32),
    mesh=_mesh,
    compiler_params=_cp,
    scratch_types=[
        pltpu.VMEM((PER_W8,), jnp.int32),
        pltpu.VMEM((PER_W8,), jnp.int32),
        pltpu.VMEM((PER_W8,), jnp.float32),
        pltpu.VMEM((PER_W9,), jnp.int32),
        pltpu.VMEM((PER_W9,), jnp.int32),
        pltpu.VMEM((PER_W9,), jnp.float32),
        pltpu.VMEM((W8, D), jnp.float32),
        pltpu.VMEM((W8, D), jnp.float32),
        pltpu.VMEM((W8, D), jnp.float32),
        pltpu.VMEM((W8, D), jnp.float32),
        pltpu.VMEM((W9, D), jnp.float32),
        pltpu.VMEM((W9, D), jnp.float32),
        pltpu.VMEM((W9, D), jnp.float32),
        pltpu.VMEM((W9, D), jnp.float32),
        pltpu.VMEM((L,), jnp.float32),
        pltpu.SemaphoreType.DMA,
        pltpu.SemaphoreType.DMA,
        pltpu.SemaphoreType.DMA,
        pltpu.SemaphoreType.DMA,
    ],
)
def _sc_loss(f8_hbm, s8_hbm, t8_hbm, w8_hbm, f9_hbm, s9_hbm, t9_hbm, w9_hbm,
             out_hbm,
             s8_v, t8_v, w8_v, s9_v, t9_v, w9_v,
             src8a_v, tgt8a_v, src8b_v, tgt8b_v,
             src9a_v, tgt9a_v, src9b_v, tgt9b_v,
             part_v, sem0, sem1, sem2, sem3):
    wid = lax.axis_index("c") * 16 + lax.axis_index("s")
    base8 = wid * PER_W8
    base9 = wid * PER_W9

    pltpu.sync_copy(s8_hbm.at[pl.ds(base8, PER_W8)], s8_v)
    pltpu.sync_copy(t8_hbm.at[pl.ds(base8, PER_W8)], t8_v)
    pltpu.sync_copy(w8_hbm.at[pl.ds(base8, PER_W8)], w8_v)
    pltpu.sync_copy(s9_hbm.at[pl.ds(base9, PER_W9)], s9_v)
    pltpu.sync_copy(t9_hbm.at[pl.ds(base9, PER_W9)], t9_v)
    pltpu.sync_copy(w9_hbm.at[pl.ds(base9, PER_W9)], w9_v)

    sems = ((sem0, sem1), (sem2, sem3))
    acc = jnp.zeros((L,), jnp.float32)
    acc = _accumulate(f8_hbm, s8_v, t8_v, w8_v,
                      ((src8a_v, tgt8a_v), (src8b_v, tgt8b_v)), sems,
                      NB8, W8, 1.0 / (2.0 * E8), acc)
    acc = _accumulate(f9_hbm, s9_v, t9_v, w9_v,
                      ((src9a_v, tgt9a_v), (src9b_v, tgt9b_v)), sems,
                      NB9, W9, 1.0 / (2.0 * E9), acc)

    part_v[...] = acc
    pltpu.sync_copy(part_v, out_hbm.at[wid])


def kernel(features_8, edge_index_8, edge_weight_8,
           features_9, edge_index_9, edge_weight_9):
    parts = _sc_loss(features_8, edge_index_8[0], edge_index_8[1], edge_weight_8,
                     features_9, edge_index_9[0], edge_index_9[1], edge_weight_9)
    return jnp.sum(parts)
